# baseline (speedup 1.0000x reference)
import functools

import jax
import jax.numpy as jnp
from jax import lax
from jax.experimental import pallas as pl
from jax.experimental.pallas import tpu as pltpu
from jax.experimental.pallas import tpu_sc as plsc

B, C, H, W = 16, 384, 24, 24
P = B * H * W
L = 16
NC, NS = 2, 16
NW = NC * NS
RPT = P // NW
BLK = 16
NBLK = RPT // BLK
NCH = C // L
DEPTH = 3


def _index_plan_kernel(bn1r_ref, bn1c_ref, bn2r_ref, bn2c_ref, th_ref,
                       g1_ref, g2_ref):
    f32 = jnp.float32
    th = th_ref[...]
    ia0 = lax.broadcasted_iota(jnp.int32, (C, C), 0)
    ia1 = lax.broadcasted_iota(jnp.int32, (C, C), 1)
    iota_row = lax.broadcasted_iota(jnp.int32, (1, C), 1)

    def plan(bn_self_r, bn_self_c, bn_other_r, bn_other_c, self_off, other_off):
        keep_r = jnp.abs(bn_self_r) >= th
        nk_c = jnp.where(jnp.abs(bn_self_c) >= th, 0.0, 1.0)
        rank = jnp.sum(jnp.where(ia0 <= ia1, jnp.broadcast_to(nk_c, (C, C)), 0.0),
                       axis=0, keepdims=True) - 1.0
        rank = jnp.clip(rank, 0.0, float(C - 1))
        ao_r = jnp.abs(bn_other_r)
        ao_c = jnp.abs(bn_other_c)
        bigger = (ao_r > ao_c) | ((ao_r == ao_c) & (ia1 < ia0))
        pos = jnp.sum(jnp.where(bigger, 1.0, 0.0), axis=1,
                      keepdims=True)
        onehot = pos == rank
        src = jnp.sum(jnp.where(onehot, ia0.astype(f32), 0.0),
                      axis=0, keepdims=True)
        return jnp.where(keep_r, iota_row + self_off,
                         src.astype(jnp.int32) + other_off)

    g1 = plan(bn1r_ref[...], bn1c_ref[...], bn2r_ref[...], bn2c_ref[...], 0, C)
    g2 = plan(bn2r_ref[...], bn2c_ref[...], bn1r_ref[...], bn1c_ref[...], C, 0)
    g1_ref[...] = g1
    g2_ref[...] = g2


def _index_plan(bn1, bn2, th):
    th_arr = jnp.asarray(th, jnp.float32).reshape(1, 1)
    out_shape = (jax.ShapeDtypeStruct((1, C), jnp.int32),) * 2
    return pl.pallas_call(_index_plan_kernel, out_shape=out_shape)(
        bn1.reshape(1, C), bn1.reshape(C, 1),
        bn2.reshape(1, C), bn2.reshape(C, 1), th_arr)


def _sc_exchange_body(f0_hbm, f1_hbm, g1_hbm, g2_hbm,
                      out1_hbm, out2_hbm, g1_v, g2_v,
                      cat0, cat1, cat2, ob1_0, ob1_1, ob1_2,
                      ob2_0, ob2_1, ob2_2,
                      insem0, insem1, insem2, osem0, osem1, osem2):
    wid = lax.axis_index("s") * NC + lax.axis_index("c")
    base = wid * RPT
    pltpu.sync_copy(g1_hbm, g1_v)
    pltpu.sync_copy(g2_hbm, g2_v)

    cats = [cat0, cat1, cat2]
    ob1s = [ob1_0, ob1_1, ob1_2]
    ob2s = [ob2_0, ob2_1, ob2_2]
    insems = [insem0, insem1, insem2]
    osems = [osem0, osem1, osem2]

    def start_in(blk, p):
        r0 = base + blk * BLK
        pltpu.async_copy(f0_hbm.at[pl.ds(r0, BLK)], cats[p].at[:, pl.ds(0, C)], insems[p])
        pltpu.async_copy(f1_hbm.at[pl.ds(r0, BLK)], cats[p].at[:, pl.ds(C, C)], insems[p])

    def wait_in(p):
        pltpu.make_async_copy(f0_hbm.at[pl.ds(0, BLK)], cats[p].at[:, pl.ds(0, C)], insems[p]).wait()
        pltpu.make_async_copy(f1_hbm.at[pl.ds(0, BLK)], cats[p].at[:, pl.ds(C, C)], insems[p]).wait()

    def start_out(blk, p):
        r0 = base + blk * BLK
        pltpu.async_copy(ob1s[p], out1_hbm.at[pl.ds(r0, BLK)], osems[p])
        pltpu.async_copy(ob2s[p], out2_hbm.at[pl.ds(r0, BLK)], osems[p])

    def wait_out(p):
        pltpu.make_async_copy(ob1s[p], out1_hbm.at[pl.ds(0, BLK)], osems[p]).wait()
        pltpu.make_async_copy(ob2s[p], out2_hbm.at[pl.ds(0, BLK)], osems[p]).wait()

    rfulls = [jnp.full((L,), r, jnp.int32) for r in range(BLK)]

    def compute(p):
        cat = cats[p]
        o1 = ob1s[p]
        o2 = ob2s[p]
        for j in range(NCH):
            gj1 = g1_v[j]
            gj2 = g2_v[j]
            gs1 = [plsc.load_gather(cat, [rfulls[r], gj1]) for r in range(BLK)]
            for r in range(BLK):
                o1[r, L * j:L * (j + 1)] = gs1[r]
            gs2 = [plsc.load_gather(cat, [rfulls[r], gj2]) for r in range(BLK)]
            for r in range(BLK):
                o2[r, L * j:L * (j + 1)] = gs2[r]

    for p in range(DEPTH):
        start_in(p, p)

    def superblock(sb, carry):
        blk0 = DEPTH * sb
        for p in range(DEPTH):
            blk = blk0 + p
            wait_in(p)

            @pl.when(sb > 0)
            def _():
                wait_out(p)

            compute(p)

            @pl.when(sb < NBLK // DEPTH - 1)
            def _():
                start_in(blk + DEPTH, p)

            start_out(blk, p)
        return carry

    lax.fori_loop(0, NBLK // DEPTH, superblock, 0)
    for p in range(DEPTH):
        wait_out(p)


@functools.lru_cache(maxsize=1)
def _sc_exchange():
    return pl.kernel(
        _sc_exchange_body,
        out_type=(jax.ShapeDtypeStruct((P, C), jnp.float32),
                  jax.ShapeDtypeStruct((P, C), jnp.float32)),
        mesh=plsc.VectorSubcoreMesh(core_axis_name="c", subcore_axis_name="s"),
        scratch_types=[
            pltpu.VMEM((NCH, L), jnp.int32),
            pltpu.VMEM((NCH, L), jnp.int32),
            pltpu.VMEM((BLK, 2 * C), jnp.float32),
            pltpu.VMEM((BLK, 2 * C), jnp.float32),
            pltpu.VMEM((BLK, 2 * C), jnp.float32),
            pltpu.VMEM((BLK, C), jnp.float32),
            pltpu.VMEM((BLK, C), jnp.float32),
            pltpu.VMEM((BLK, C), jnp.float32),
            pltpu.VMEM((BLK, C), jnp.float32),
            pltpu.VMEM((BLK, C), jnp.float32),
            pltpu.VMEM((BLK, C), jnp.float32),
            pltpu.SemaphoreType.DMA,
            pltpu.SemaphoreType.DMA,
            pltpu.SemaphoreType.DMA,
            pltpu.SemaphoreType.DMA,
            pltpu.SemaphoreType.DMA,
            pltpu.SemaphoreType.DMA,
        ],
        compiler_params=pltpu.CompilerParams(needs_layout_passes=False),
    )


def kernel(features_0, features_1, bn1_weight, bn2_weight, bn_threshold):
    g1, g2 = _index_plan(bn1_weight, bn2_weight, bn_threshold)
    f0 = features_0.transpose(0, 2, 3, 1).reshape(P, C)
    f1 = features_1.transpose(0, 2, 3, 1).reshape(P, C)
    o1, o2 = _sc_exchange()(f0, f1, g1.reshape(NCH, L), g2.reshape(NCH, L))
    out1 = o1.reshape(B, H, W, C).transpose(0, 3, 1, 2)
    out2 = o2.reshape(B, H, W, C).transpose(0, 3, 1, 2)
    return (out1, out2)
